# Initial kernel scaffold; baseline (speedup 1.0000x reference)
import jax
import jax.numpy as jnp
from jax.experimental import pallas as pl


def kernel(x, w1, b1, w3, b3, gn_w, gn_b):
    raise NotImplementedError("write your pallas kernel here")


# speedup vs baseline: 2.4068x; 2.4068x over previous
import functools

import numpy as np

import jax
import jax.numpy as jnp
from jax import lax
from jax.experimental import pallas as pl
from jax.experimental.pallas import tpu as pltpu

_GROUPS = 8


def _ema_body(x_ref, pool_ref, rt_ref, ct_ref, w1bd_ref, b1c_ref, w3all_ref,
              b3c_ref, gwc_ref, gbc_ref, o_ref, stack_ref, *, cg, h, w):
    f32 = jnp.float32
    S = cg * _GROUPS
    HW = h * w

    xs = x_ref[0]

    pooled = jnp.dot(xs, pool_ref[...], preferred_element_type=f32)
    att = jax.nn.sigmoid(
        jnp.dot(w1bd_ref[...], pooled, preferred_element_type=f32) + b1c_ref[...])
    xh_full = jnp.dot(att[:, :h], rt_ref[...], preferred_element_type=f32)
    xw_full = jnp.dot(att[:, h:], ct_ref[...], preferred_element_type=f32)

    v = xs * xh_full * xw_full
    mu = jnp.mean(v, axis=-1, keepdims=True)
    msq = jnp.mean(v * v, axis=-1, keepdims=True)
    var = jnp.maximum(msq - mu * mu, 0.0)
    x1 = (v - mu) * lax.rsqrt(var + 1e-5) * gwc_ref[...] + gbc_ref[...]

    row1 = lax.broadcasted_iota(jnp.int32, (1, HW), 1) // w
    col1 = lax.broadcasted_iota(jnp.int32, (1, HW), 1) % w
    t = 0
    for dy in (-1, 0, 1):
        for dx in (-1, 0, 1):
            if dy == 0 and dx == 0:
                sh = xs
            else:
                sh = jnp.roll(xs, -(dy * w + dx), axis=1)
                mask = None
                if dy != 0:
                    mask = (row1 + dy >= 0) & (row1 + dy < h)
                if dx != 0:
                    m = (col1 + dx >= 0) & (col1 + dx < w)
                    mask = m if mask is None else (mask & m)
                sh = jnp.where(mask, sh, 0.0)
            stack_ref[t * S:(t + 1) * S, :] = sh
            t += 1
    x2 = (jnp.dot(w3all_ref[...], stack_ref[...], preferred_element_type=f32)
          + b3c_ref[...])

    m1 = jnp.mean(x1, axis=-1, keepdims=True)
    m2 = jnp.mean(x2, axis=-1, keepdims=True)
    r_idx = lax.broadcasted_iota(jnp.int32, (S, 1), 0)

    def butterfly(col, op):
        k = 1
        while k < cg:
            down = jnp.roll(col, -k, axis=0)
            up = jnp.roll(col, k, axis=0)
            partner = jnp.where((r_idx // k) % 2 == 0, down, up)
            col = op(col, partner)
            k *= 2
        return col

    def gsoftmax(m):
        mx = butterfly(m, jnp.maximum)
        e = jnp.exp(m - mx)
        return e / butterfly(e, jnp.add)

    a1 = gsoftmax(m1)
    a2 = gsoftmax(m2)
    pre = a1 * x2 + a2 * x1

    gi = lax.broadcasted_iota(jnp.int32, (S, S), 0) // cg
    gj = lax.broadcasted_iota(jnp.int32, (S, S), 1) // cg
    grp = (gi == gj).astype(f32)
    wts = jnp.dot(grp, pre, preferred_element_type=f32)

    o_ref[0] = (xs * jax.nn.sigmoid(wts)).astype(o_ref.dtype)


def _pool_consts(h, w, dtype=np.float32):
    hw = h * w
    p = np.arange(hw)
    q = np.arange(h + w)
    prow, pcol = p // w, p % w
    pool = np.where(q[None, :] < h,
                    (prow[:, None] == q[None, :]) / w,
                    (pcol[:, None] == (q[None, :] - h)) / h).astype(dtype)
    rt = (np.arange(h)[:, None] == prow[None, :]).astype(dtype)
    ct = (np.arange(w)[:, None] == pcol[None, :]).astype(dtype)
    return pool, rt, ct


@jax.jit
def kernel(x, w1, b1, w3, b3, gn_w, gn_b):
    b, c, h, w = x.shape
    cg = c // _GROUPS
    S = c
    hw = h * w
    f32 = jnp.float32

    x3 = x.reshape(b, S, hw)

    eye = jnp.eye(_GROUPS, dtype=f32)
    w1bd = jnp.kron(eye, w1[:, :, 0, 0])
    w3t = w3.transpose(2, 3, 0, 1).reshape(9, cg, cg)
    w3blocks = jax.vmap(lambda m: jnp.kron(eye, m))(w3t)
    w3all = w3blocks.transpose(1, 0, 2).reshape(S, 9 * S)
    b1c = jnp.tile(b1, _GROUPS).reshape(S, 1)
    b3c = jnp.tile(b3, _GROUPS).reshape(S, 1)
    gwc = jnp.tile(gn_w, _GROUPS).reshape(S, 1)
    gbc = jnp.tile(gn_b, _GROUPS).reshape(S, 1)

    pool_np, rt_np, ct_np = _pool_consts(h, w)
    pool = jnp.asarray(pool_np)
    rt = jnp.asarray(rt_np)
    ct = jnp.asarray(ct_np)

    body = functools.partial(_ema_body, cg=cg, h=h, w=w)
    const = lambda shape: pl.BlockSpec(shape, lambda i: (0,) * len(shape))

    out = pl.pallas_call(
        body,
        out_shape=jax.ShapeDtypeStruct((b, S, hw), x.dtype),
        grid=(b,),
        in_specs=[
            pl.BlockSpec((1, S, hw), lambda i: (i, 0, 0)),
            const((hw, h + w)),
            const((h, hw)),
            const((w, hw)),
            const((S, S)),
            const((S, 1)),
            const((S, 9 * S)),
            const((S, 1)),
            const((S, 1)),
            const((S, 1)),
        ],
        out_specs=pl.BlockSpec((1, S, hw), lambda i: (i, 0, 0)),
        scratch_shapes=[pltpu.VMEM((9 * S, hw), f32)],
        compiler_params=pltpu.CompilerParams(
            dimension_semantics=("parallel",),
        ),
    )(x3, pool, rt, ct, w1bd, b1c, w3all, b3c, gwc, gbc)

    return out.reshape(b, c, h, w)


# speedup vs baseline: 2.4119x; 1.0021x over previous
import functools

import numpy as np

import jax
import jax.numpy as jnp
from jax import lax
from jax.experimental import pallas as pl
from jax.experimental.pallas import tpu as pltpu

_GROUPS = 8


def _ema_body(x_ref, pool_ref, rtct_ref, w1bd_ref, b1c_ref,
              w3all_ref, b3c_ref, gwc_ref, gbc_ref, o_ref, stack_ref,
              *, cg, h, w):
    f32 = jnp.float32
    S = cg * _GROUPS
    HW = h * w

    xs = x_ref[0]

    pooled = jnp.dot(xs, pool_ref[...], preferred_element_type=f32)
    att = jax.nn.sigmoid(
        jnp.dot(w1bd_ref[...], pooled, preferred_element_type=f32) + b1c_ref[...])
    both = jnp.dot(att, rtct_ref[...], preferred_element_type=f32)
    xh_full = both[:, :HW]
    xw_full = both[:, HW:]

    v = xs * xh_full * xw_full
    mu = jnp.mean(v, axis=-1, keepdims=True)
    msq = jnp.mean(v * v, axis=-1, keepdims=True)
    var = jnp.maximum(msq - mu * mu, 0.0)
    s1 = lax.rsqrt(var + 1e-5) * gwc_ref[...]
    s2 = gbc_ref[...] - mu * s1
    x1 = v * s1 + s2

    row1 = lax.broadcasted_iota(jnp.int32, (1, HW), 1) // w
    col1 = lax.broadcasted_iota(jnp.int32, (1, HW), 1) % w
    t = 0
    for dy in (-1, 0, 1):
        for dx in (-1, 0, 1):
            if dy == 0 and dx == 0:
                sh = xs
            else:
                sh = jnp.roll(xs, -(dy * w + dx), axis=1)
                mask = None
                if dy != 0:
                    mask = (row1 + dy >= 0) & (row1 + dy < h)
                if dx != 0:
                    m = (col1 + dx >= 0) & (col1 + dx < w)
                    mask = m if mask is None else (mask & m)
                sh = jnp.where(mask, sh, 0.0)
            stack_ref[t * S:(t + 1) * S, :] = sh
            t += 1
    x2 = (jnp.dot(w3all_ref[...], stack_ref[...], preferred_element_type=f32)
          + b3c_ref[...])

    m1 = jnp.mean(x1, axis=-1, keepdims=True)
    m2 = jnp.mean(x2, axis=-1, keepdims=True)
    r_idx = lax.broadcasted_iota(jnp.int32, (S, 1), 0)

    def butterfly(col, op):
        k = 1
        while k < cg:
            down = jnp.roll(col, -k, axis=0)
            up = jnp.roll(col, k, axis=0)
            partner = jnp.where((r_idx // k) % 2 == 0, down, up)
            col = op(col, partner)
            k *= 2
        return col

    def gsoftmax(m):
        mx = butterfly(m, jnp.maximum)
        e = jnp.exp(m - mx)
        return e / butterfly(e, jnp.add)

    a1 = gsoftmax(m1)
    a2 = gsoftmax(m2)
    pre = a1 * x2 + a2 * x1

    gi = lax.broadcasted_iota(jnp.int32, (S, S), 0) // cg
    gj = lax.broadcasted_iota(jnp.int32, (S, S), 1) // cg
    grp = (gi == gj).astype(f32)
    wts = jnp.dot(grp, pre, preferred_element_type=f32)

    o_ref[0] = (xs * jax.nn.sigmoid(wts)).astype(o_ref.dtype)


def _pool_consts(h, w, dtype=np.float32):
    hw = h * w
    p = np.arange(hw)
    q = np.arange(h + w)
    prow, pcol = p // w, p % w
    pool = np.where(q[None, :] < h,
                    (prow[:, None] == q[None, :]) / w,
                    (pcol[:, None] == (q[None, :] - h)) / h).astype(dtype)
    rt = (np.arange(h)[:, None] == prow[None, :]).astype(dtype)
    ct = (np.arange(w)[:, None] == pcol[None, :]).astype(dtype)
    rtct = np.zeros((h + w, 2 * hw), dtype)
    rtct[:h, :hw] = rt
    rtct[h:, hw:] = ct
    masks = []
    for dy in (-1, 0, 1):
        for dx in (-1, 0, 1):
            if dy == 0 and dx == 0:
                continue
            m = np.ones(hw, dtype=bool)
            if dy != 0:
                m &= (prow + dy >= 0) & (prow + dy < h)
            if dx != 0:
                m &= (pcol + dx >= 0) & (pcol + dx < w)
            masks.append(m)
    mask8 = np.stack(masks).astype(dtype)
    return pool, rtct, mask8


@jax.jit
def kernel(x, w1, b1, w3, b3, gn_w, gn_b):
    b, c, h, w = x.shape
    cg = c // _GROUPS
    S = c
    hw = h * w
    f32 = jnp.float32

    x3 = x.reshape(b, S, hw)

    eye = jnp.eye(_GROUPS, dtype=f32)
    w1bd = jnp.kron(eye, w1[:, :, 0, 0])
    w3t = w3.transpose(2, 3, 0, 1).reshape(9, cg, cg)
    w3blocks = jax.vmap(lambda m: jnp.kron(eye, m))(w3t)
    w3all = w3blocks.transpose(1, 0, 2).reshape(S, 9 * S)
    b1c = jnp.tile(b1, _GROUPS).reshape(S, 1)
    b3c = jnp.tile(b3, _GROUPS).reshape(S, 1)
    gwc = jnp.tile(gn_w, _GROUPS).reshape(S, 1)
    gbc = jnp.tile(gn_b, _GROUPS).reshape(S, 1)

    pool_np, rtct_np, _ = _pool_consts(h, w)
    pool = jnp.asarray(pool_np)
    rtct = jnp.asarray(rtct_np)

    body = functools.partial(_ema_body, cg=cg, h=h, w=w)
    const = lambda shape: pl.BlockSpec(shape, lambda i: (0,) * len(shape))

    out = pl.pallas_call(
        body,
        out_shape=jax.ShapeDtypeStruct((b, S, hw), x.dtype),
        grid=(b,),
        in_specs=[
            pl.BlockSpec((1, S, hw), lambda i: (i, 0, 0)),
            const((hw, h + w)),
            const((h + w, 2 * hw)),
            const((S, S)),
            const((S, 1)),
            const((S, 9 * S)),
            const((S, 1)),
            const((S, 1)),
            const((S, 1)),
        ],
        out_specs=pl.BlockSpec((1, S, hw), lambda i: (i, 0, 0)),
        scratch_shapes=[pltpu.VMEM((9 * S, hw), f32)],
        compiler_params=pltpu.CompilerParams(
            dimension_semantics=("parallel",),
        ),
    )(x3, pool, rtct, w1bd, b1c, w3all, b3c, gwc, gbc)

    return out.reshape(b, c, h, w)


# speedup vs baseline: 3.8976x; 1.6160x over previous
import functools

import numpy as np

import jax
import jax.numpy as jnp
from jax import lax
from jax.experimental import pallas as pl
from jax.experimental.pallas import tpu as pltpu

_GROUPS = 8


def _ema_body(x_ref, pool_ref, rtct_ref, w1bd_ref, b1c_ref,
              w3all_ref, b3c_ref, gwc_ref, gbc_ref, o_ref, stack_ref,
              *, cg, h, w):
    f32 = jnp.float32
    S = cg * _GROUPS
    HW = h * w

    xs = x_ref[0].reshape(S, HW)

    pooled = jnp.dot(xs, pool_ref[...], preferred_element_type=f32)
    att = jax.nn.sigmoid(
        jnp.dot(w1bd_ref[...], pooled, preferred_element_type=f32) + b1c_ref[...])
    both = jnp.dot(att, rtct_ref[...], preferred_element_type=f32)
    xh_full = both[:, :HW]
    xw_full = both[:, HW:]

    v = xs * xh_full * xw_full
    mu = jnp.mean(v, axis=-1, keepdims=True)
    msq = jnp.mean(v * v, axis=-1, keepdims=True)
    var = jnp.maximum(msq - mu * mu, 0.0)
    s1 = lax.rsqrt(var + 1e-5) * gwc_ref[...]
    s2 = gbc_ref[...] - mu * s1
    x1 = v * s1 + s2

    row1 = lax.broadcasted_iota(jnp.int32, (1, HW), 1) // w
    col1 = lax.broadcasted_iota(jnp.int32, (1, HW), 1) % w
    t = 0
    for dy in (-1, 0, 1):
        for dx in (-1, 0, 1):
            if dy == 0 and dx == 0:
                sh = xs
            else:
                sh = jnp.roll(xs, -(dy * w + dx), axis=1)
                mask = None
                if dy != 0:
                    mask = (row1 + dy >= 0) & (row1 + dy < h)
                if dx != 0:
                    m = (col1 + dx >= 0) & (col1 + dx < w)
                    mask = m if mask is None else (mask & m)
                sh = jnp.where(mask, sh, 0.0)
            stack_ref[t * S:(t + 1) * S, :] = sh
            t += 1
    x2 = (jnp.dot(w3all_ref[...], stack_ref[...], preferred_element_type=f32)
          + b3c_ref[...])

    m1 = jnp.mean(x1, axis=-1, keepdims=True)
    m2 = jnp.mean(x2, axis=-1, keepdims=True)
    r_idx = lax.broadcasted_iota(jnp.int32, (S, 1), 0)

    def butterfly(col, op):
        k = 1
        while k < cg:
            down = jnp.roll(col, -k, axis=0)
            up = jnp.roll(col, k, axis=0)
            partner = jnp.where((r_idx // k) % 2 == 0, down, up)
            col = op(col, partner)
            k *= 2
        return col

    def gsoftmax(m):
        mx = butterfly(m, jnp.maximum)
        e = jnp.exp(m - mx)
        return e / butterfly(e, jnp.add)

    a1 = gsoftmax(m1)
    a2 = gsoftmax(m2)
    pre = a1 * x2 + a2 * x1

    gi = lax.broadcasted_iota(jnp.int32, (S, S), 0) // cg
    gj = lax.broadcasted_iota(jnp.int32, (S, S), 1) // cg
    grp = (gi == gj).astype(f32)
    wts = jnp.dot(grp, pre, preferred_element_type=f32)

    o_ref[0] = (xs * jax.nn.sigmoid(wts)).astype(o_ref.dtype).reshape(S, h, w)


def _pool_consts(h, w, dtype=np.float32):
    hw = h * w
    p = np.arange(hw)
    q = np.arange(h + w)
    prow, pcol = p // w, p % w
    pool = np.where(q[None, :] < h,
                    (prow[:, None] == q[None, :]) / w,
                    (pcol[:, None] == (q[None, :] - h)) / h).astype(dtype)
    rt = (np.arange(h)[:, None] == prow[None, :]).astype(dtype)
    ct = (np.arange(w)[:, None] == pcol[None, :]).astype(dtype)
    rtct = np.zeros((h + w, 2 * hw), dtype)
    rtct[:h, :hw] = rt
    rtct[h:, hw:] = ct
    masks = []
    for dy in (-1, 0, 1):
        for dx in (-1, 0, 1):
            if dy == 0 and dx == 0:
                continue
            m = np.ones(hw, dtype=bool)
            if dy != 0:
                m &= (prow + dy >= 0) & (prow + dy < h)
            if dx != 0:
                m &= (pcol + dx >= 0) & (pcol + dx < w)
            masks.append(m)
    mask8 = np.stack(masks).astype(dtype)
    return pool, rtct, mask8


@jax.jit
def kernel(x, w1, b1, w3, b3, gn_w, gn_b):
    b, c, h, w = x.shape
    cg = c // _GROUPS
    S = c
    hw = h * w
    f32 = jnp.float32

    eye = jnp.eye(_GROUPS, dtype=f32)
    w1bd = jnp.kron(eye, w1[:, :, 0, 0])
    w3t = w3.transpose(2, 3, 0, 1).reshape(9, cg, cg)
    w3blocks = jax.vmap(lambda m: jnp.kron(eye, m))(w3t)
    w3all = w3blocks.transpose(1, 0, 2).reshape(S, 9 * S)
    b1c = jnp.tile(b1, _GROUPS).reshape(S, 1)
    b3c = jnp.tile(b3, _GROUPS).reshape(S, 1)
    gwc = jnp.tile(gn_w, _GROUPS).reshape(S, 1)
    gbc = jnp.tile(gn_b, _GROUPS).reshape(S, 1)

    pool_np, rtct_np, _ = _pool_consts(h, w)
    pool = jnp.asarray(pool_np)
    rtct = jnp.asarray(rtct_np)

    body = functools.partial(_ema_body, cg=cg, h=h, w=w)
    const = lambda shape: pl.BlockSpec(shape, lambda i: (0,) * len(shape))

    out = pl.pallas_call(
        body,
        out_shape=jax.ShapeDtypeStruct((b, S, h, w), x.dtype),
        grid=(b,),
        in_specs=[
            pl.BlockSpec((1, S, h, w), lambda i: (i, 0, 0, 0)),
            const((hw, h + w)),
            const((h + w, 2 * hw)),
            const((S, S)),
            const((S, 1)),
            const((S, 9 * S)),
            const((S, 1)),
            const((S, 1)),
            const((S, 1)),
        ],
        out_specs=pl.BlockSpec((1, S, h, w), lambda i: (i, 0, 0, 0)),
        scratch_shapes=[pltpu.VMEM((9 * S, hw), f32)],
        compiler_params=pltpu.CompilerParams(
            dimension_semantics=("parallel",),
        ),
    )(x, pool, rtct, w1bd, b1c, w3all, b3c, gwc, gbc)

    return out
